# baseline (speedup 1.0000x reference)
import functools
import math

import jax
import jax.numpy as jnp
from jax import lax
from jax.experimental import pallas as pl
from jax.experimental.pallas import tpu as pltpu
from jax.experimental.pallas import tpu_sc as plsc

N_NODES = 1000000
N_DIM = 16
R = 10.0
BETA = 1.0
SIGMA = 1.0
BATCH = 16384

_CONST = 0.5 * N_DIM * math.log(2.0 * math.pi * SIGMA**2)
_LATENT_SCALE = 1.0 / (N_NODES - 1)

NC = 2
NS = 16
NW = NC * NS
B_PER_W = BATCH // NW
R_PER_W = 2 * B_PER_W
CHUNK = 128
N_CHUNK = R_PER_W // CHUNK
GROUPS = N_NODES // 8


def _sc_body(pidx_hbm, tgr_hbm, s_hbm, t_hbm,
             pidx_v, gidx_v, buf0, buf1, s_v, t_v, sem0, sem1):
    wid = lax.axis_index("s") * NC + lax.axis_index("c")
    base = wid * B_PER_W
    bufs = (buf0, buf1)
    sems = (sem0, sem1)

    pltpu.sync_copy(pidx_hbm.at[wid], pidx_v)
    for c in range(N_CHUNK):
        for q in range(CHUNK // 16):
            sl = pl.ds(q * 16, 16)
            gidx_v[c, sl] = lax.shift_right_logical(pidx_v[c, sl], 3)

    def fire(c):
        return pltpu.async_copy(tgr_hbm.at[gidx_v.at[c]], bufs[c % 2],
                                sems[c % 2])

    def compute(c):
        buf = bufs[c % 2]

        def blk_body(b, _, c=c, buf=buf):
            eu = 2 * pl.multiple_of(b * N_DIM, N_DIM) + 2 * lax.iota(
                jnp.int32, 16)
            ev = eu + 1
            crow = jnp.full((16,), c, jnp.int32)
            cu = 16 * (plsc.load_gather(pidx_v, [crow, eu]) & 7)
            cv = 16 * (plsc.load_gather(pidx_v, [crow, ev]) & 7)
            acc = jnp.zeros((16,), jnp.float32)
            tot = jnp.zeros((16,), jnp.float32)
            for d in range(N_DIM):
                uc = plsc.load_gather(buf, [eu, cu + d])
                vc = plsc.load_gather(buf, [ev, cv + d])
                df = uc - vc
                acc = acc + df * df
                tot = tot + uc * uc + vc * vc
            out = pl.ds(c * (CHUNK // 2) + pl.multiple_of(b * N_DIM, N_DIM), 16)
            s_v[out] = acc
            t_v[out] = tot
            return 0

        lax.fori_loop(0, CHUNK // (2 * N_DIM), blk_body, 0)

    cp = [fire(0), fire(1)]
    for c in range(N_CHUNK):
        cp[c % 2].wait()
        compute(c)
        if c + 2 < N_CHUNK:
            cp[c % 2] = fire(c + 2)

    pltpu.sync_copy(s_v, s_hbm.at[pl.ds(base, B_PER_W)])
    pltpu.sync_copy(t_v, t_hbm.at[pl.ds(base, B_PER_W)])


_sc_reduce = functools.partial(
    pl.kernel,
    out_type=(jax.ShapeDtypeStruct((BATCH,), jnp.float32),
              jax.ShapeDtypeStruct((BATCH,), jnp.float32)),
    mesh=plsc.VectorSubcoreMesh(core_axis_name="c", subcore_axis_name="s"),
    compiler_params=pltpu.CompilerParams(
        needs_layout_passes=False, use_tc_tiling_on_sc=True),
    scratch_types=[
        pltpu.VMEM((N_CHUNK, CHUNK), jnp.int32),
        pltpu.VMEM((N_CHUNK, CHUNK), jnp.int32),
        pltpu.VMEM((CHUNK, 128), jnp.float32),
        pltpu.VMEM((CHUNK, 128), jnp.float32),
        pltpu.VMEM((B_PER_W,), jnp.float32),
        pltpu.VMEM((B_PER_W,), jnp.float32),
        pltpu.SemaphoreType.DMA,
        pltpu.SemaphoreType.DMA,
    ],
)(_sc_body)


def _tc_math_body(s_ref, t_ref, y_ref, o_ref):
    s = s_ref[...]
    t = t_ref[...]
    y = y_ref[...].astype(jnp.float32)
    dist = jnp.sqrt(s)
    x = BETA * (dist - R)
    softplus = jnp.log1p(jnp.exp(-jnp.abs(x)))
    pair = y * jnp.maximum(x, 0.0) + (1.0 - y) * jnp.maximum(-x, 0.0) + softplus
    o_ref[...] = pair + (0.5 * t + 2.0 * _CONST) * _LATENT_SCALE


def kernel(pairs, labels, table):
    pidx = pairs.astype(jnp.int32).reshape(NW, N_CHUNK, CHUNK)
    tgroups = table.reshape(GROUPS, 128)
    s, t = _sc_reduce(pidx, tgroups)
    loss = pl.pallas_call(
        _tc_math_body,
        out_shape=jax.ShapeDtypeStruct((128, 128), jnp.float32),
    )(s.reshape(128, 128), t.reshape(128, 128), labels.reshape(128, 128))
    return loss.reshape(BATCH)


# speedup vs baseline: 1.5667x; 1.5667x over previous
import functools
import math

import jax
import jax.numpy as jnp
from jax import lax
from jax.experimental import pallas as pl
from jax.experimental.pallas import tpu as pltpu
from jax.experimental.pallas import tpu_sc as plsc

N_NODES = 1000000
N_DIM = 16
R = 10.0
BETA = 1.0
SIGMA = 1.0
BATCH = 16384

_CONST = 0.5 * N_DIM * math.log(2.0 * math.pi * SIGMA**2)
_LATENT_SCALE = 1.0 / (N_NODES - 1)

NC = 2
NS = 16
NW = NC * NS
B_PER_W = BATCH // NW
R_PER_W = 2 * B_PER_W
CHUNK = 128
N_CHUNK = R_PER_W // CHUNK


def _sc_body(pidx_hbm, table_hbm, s_hbm, t_hbm,
             idx_v, buf0, buf1, s_v, t_v, sems):
    wid = lax.axis_index("s") * NC + lax.axis_index("c")
    base = wid * B_PER_W
    bufs = (buf0, buf1)

    pltpu.sync_copy(pidx_hbm.at[wid], idx_v)

    def fire(c):
        buf = bufs[c % 2]

        def fire_q(q, _, c=c, buf=buf):
            off = pl.multiple_of(q * 16, 16)
            vec = idx_v[c, pl.ds(off, 16)]
            for k in range(16):
                pltpu.async_copy(
                    table_hbm.at[pl.ds(vec[k], 1)],
                    buf.at[pl.ds(off + k, 1)],
                    sems.at[c])
            return 0
        lax.fori_loop(0, CHUNK // 16, fire_q, 0)

    def drain(c):
        pltpu.make_async_copy(
            table_hbm.at[pl.ds(0, CHUNK)], bufs[c % 2], sems.at[c]).wait()

    def compute(c):
        buf = bufs[c % 2]

        def blk_body(b, _, c=c, buf=buf):
            eu = 2 * pl.multiple_of(b * N_DIM, N_DIM) + 2 * lax.iota(
                jnp.int32, 16)
            ev = eu + 1
            acc = jnp.zeros((16,), jnp.float32)
            tot = jnp.zeros((16,), jnp.float32)
            for d in range(N_DIM):
                col = jnp.full((16,), d, jnp.int32)
                uc = plsc.load_gather(buf, [eu, col])
                vc = plsc.load_gather(buf, [ev, col])
                df = uc - vc
                acc = acc + df * df
                tot = tot + uc * uc + vc * vc
            out = pl.ds(c * (CHUNK // 2) + pl.multiple_of(b * N_DIM, N_DIM), 16)
            s_v[out] = acc
            t_v[out] = tot
            return 0

        lax.fori_loop(0, CHUNK // (2 * N_DIM), blk_body, 0)

    fire(0)
    fire(1)
    for c in range(N_CHUNK):
        drain(c)
        compute(c)
        if c + 2 < N_CHUNK:
            fire(c + 2)

    pltpu.sync_copy(s_v, s_hbm.at[pl.ds(base, B_PER_W)])
    pltpu.sync_copy(t_v, t_hbm.at[pl.ds(base, B_PER_W)])


_sc_reduce = functools.partial(
    pl.kernel,
    out_type=(jax.ShapeDtypeStruct((BATCH,), jnp.float32),
              jax.ShapeDtypeStruct((BATCH,), jnp.float32)),
    mesh=plsc.VectorSubcoreMesh(core_axis_name="c", subcore_axis_name="s"),
    compiler_params=pltpu.CompilerParams(
        needs_layout_passes=False, use_tc_tiling_on_sc=True),
    scratch_types=[
        pltpu.VMEM((N_CHUNK, CHUNK), jnp.int32),
        pltpu.VMEM((CHUNK, N_DIM), jnp.float32),
        pltpu.VMEM((CHUNK, N_DIM), jnp.float32),
        pltpu.VMEM((B_PER_W,), jnp.float32),
        pltpu.VMEM((B_PER_W,), jnp.float32),
        pltpu.SemaphoreType.DMA((N_CHUNK,)),
    ],
)(_sc_body)


def _tc_math_body(s_ref, t_ref, y_ref, o_ref):
    s = s_ref[...]
    t = t_ref[...]
    y = y_ref[...].astype(jnp.float32)
    dist = jnp.sqrt(s)
    x = BETA * (dist - R)
    softplus = jnp.log1p(jnp.exp(-jnp.abs(x)))
    pair = y * jnp.maximum(x, 0.0) + (1.0 - y) * jnp.maximum(-x, 0.0) + softplus
    o_ref[...] = pair + (0.5 * t + 2.0 * _CONST) * _LATENT_SCALE


def kernel(pairs, labels, table):
    pidx = pairs.astype(jnp.int32).reshape(NW, N_CHUNK, CHUNK)
    s, t = _sc_reduce(pidx, table)
    loss = pl.pallas_call(
        _tc_math_body,
        out_shape=jax.ShapeDtypeStruct((128, 128), jnp.float32),
    )(s.reshape(128, 128), t.reshape(128, 128), labels.reshape(128, 128))
    return loss.reshape(BATCH)
